# baseline (speedup 1.0000x reference)
import functools

import jax
import jax.numpy as jnp
from jax import lax
from jax.experimental import pallas as pl
from jax.experimental.pallas import tpu as pltpu
from jax.experimental.pallas import tpu_sc as plsc

NC = 2
NS = 16
ECH = 256
CS = 40


@functools.lru_cache(maxsize=None)
def _build(n_nodes, n_edges, half, n_user, n_layers):
    edges_pt = n_edges // NS
    assert n_edges % (NS * ECH) == 0
    nfull = edges_pt // ECH
    assert n_nodes % CS == 0 and n_user % CS == 0
    nchunk = n_nodes // CS
    per = nchunk // NS
    extra = nchunk % NS
    ucut = n_user // CS
    nbuf = n_layers - 1

    mesh = plsc.VectorSubcoreMesh(
        core_axis_name="c", subcore_axis_name="s",
        num_cores=NC, num_subcores=NS)

    out_type = (
        jax.ShapeDtypeStruct((NC, n_user, half), jnp.float32),
        jax.ShapeDtypeStruct((NC, n_nodes - n_user, half), jnp.float32),
        jax.ShapeDtypeStruct((nbuf, NC, n_nodes, half), jnp.float32),
    )
    assert nfull % 3 == 0
    scratch = []
    for _ in range(3):
        scratch += [
            pltpu.VMEM((ECH,), jnp.int32),
            pltpu.VMEM((ECH // 128, 128), jnp.int32),
            pltpu.VMEM((ECH,), jnp.float32),
            pltpu.VMEM((ECH, half), jnp.float32),
            pltpu.SemaphoreType.DMA,
            pltpu.SemaphoreType.DMA,
            pltpu.SemaphoreType.DMA,
        ]
    scratch += [
        pltpu.VMEM_SHARED((n_nodes, half), jnp.float32),
        pltpu.SemaphoreType.DMA,
    ]

    @functools.partial(pl.kernel, out_type=out_type, mesh=mesh,
                       scratch_types=scratch,
                       compiler_params=pltpu.CompilerParams(
                           use_tc_tiling_on_sc=False))
    def k(x0f, x0t, zer, srch, dst2, wh, user_o, item_o, xbuf, *scr):
        bufs = [tuple(scr[i * 7:(i + 1) * 7]) for i in range(3)]
        (acc, sem) = scr[21:]
        b0 = bufs[0][3].at[pl.ds(0, CS)]
        b1 = bufs[1][3].at[pl.ds(0, CS)]
        b2 = bufs[2][3].at[pl.ds(0, CS)]
        b3 = bufs[0][3].at[pl.ds(CS + 8, CS)]
        c = lax.axis_index("c")
        s = lax.axis_index("s")
        ebase = s * edges_pt
        my_rows = per + jnp.where(s < extra, 1, 0)

        def row_loop(body):
            def rb(j, _):
                body(s + j * NS)
                return 0
            lax.fori_loop(0, my_rows, rb, 0)

        def transform_idx(sv_, size):
            def tb(i, _):
                sl = pl.ds(i * 16, 16)
                sv_[sl] = sv_[sl] * 2 + c
                return 0
            lax.fori_loop(0, size // 16, tb, 0, unroll=True)

        def scale(wv_, rv_, size):
            def sc_body(g, _):
                w16 = wv_[pl.ds(g * 16, 16)]
                for l in range(16):
                    e = g * 16 + l
                    we = w16[l]
                    for h in range(0, half, 16):
                        ix = (e, pl.ds(h, 16))
                        rv_[ix] = rv_[ix] * we
                return 0
            lax.fori_loop(0, size // 16, sc_body, 0)

        def edge_phase(layer, src_ref):
            nsub = ECH // 128

            def fire_idx(t, B):
                base = ebase + t * ECH
                pltpu.async_copy(srch.at[pl.ds(base, ECH)], B[0], B[6])
                pltpu.async_copy(dst2.at[pl.ds((ebase + t * ECH) // 128, nsub)],
                                 B[1], B[6])
                pltpu.async_copy(wh.at[pl.ds(base, ECH)], B[2], B[6])

            def wait_idx(t, B):
                base = ebase + t * ECH
                pltpu.make_async_copy(srch.at[pl.ds(base, ECH)], B[0], B[6]).wait()
                pltpu.make_async_copy(dst2.at[pl.ds((ebase + t * ECH) // 128, nsub)],
                                      B[1], B[6]).wait()
                pltpu.make_async_copy(wh.at[pl.ds(base, ECH)], B[2], B[6]).wait()

            def start_gather(B):
                if layer == 0:
                    transform_idx(B[0], ECH)
                for q in range(nsub):
                    pltpu.async_copy(src_ref.at[B[0].at[pl.ds(q * 128, 128)]],
                                     B[3].at[pl.ds(q * 128, 128)], B[4])

            def wait_gather(B):
                for q in range(nsub):
                    pltpu.make_async_copy(src_ref.at[B[0].at[pl.ds(q * 128, 128)]],
                                          B[3].at[pl.ds(q * 128, 128)], B[4]).wait()

            def fire_scatter(B):
                for q in range(nsub):
                    pltpu.async_copy(B[3].at[pl.ds(q * 128, 128)],
                                     acc.at[B[1].at[q]], B[5], add=True)

            def wait_scatter(B):
                for q in range(nsub):
                    pltpu.make_async_copy(B[3].at[pl.ds(q * 128, 128)],
                                          acc.at[B[1].at[q]], B[5]).wait()

            for t0 in range(2):
                B = bufs[t0]
                fire_idx(t0, B)
                wait_idx(t0, B)
                start_gather(B)

            def body(j, _):
                for kk in range(3):
                    B = bufs[kk]
                    B2 = bufs[(kk + 2) % 3]
                    t = 3 * j + kk
                    wait_gather(B)

                    @pl.when(t >= 1)
                    def _():
                        wait_scatter(B2)

                    @pl.when(t + 2 < nfull)
                    def _():
                        fire_idx(t + 2, B2)
                    scale(B[2], B[3], ECH)
                    fire_scatter(B)

                    @pl.when(t + 2 < nfull)
                    def _():
                        wait_idx(t + 2, B2)
                        start_gather(B2)
                return 0
            lax.fori_loop(0, nfull // 3, body, 0)
            wait_scatter(bufs[2])

        row_loop(lambda cid: pltpu.sync_copy(zer, acc.at[pl.ds(cid * CS, CS)]))

        for layer in range(n_layers):
            plsc.subcore_barrier()
            edge_phase(layer, x0f if layer == 0 else xbuf.at[layer - 1, c])
            plsc.subcore_barrier()
            if layer < n_layers - 1:
                def wb(cid):
                    rs = pl.ds(cid * CS, CS)
                    pltpu.sync_copy(acc.at[rs], xbuf.at[layer, c, rs])
                    pltpu.sync_copy(zer, acc.at[rs])
                row_loop(wb)
                plsc.subcore_barrier()

        def fm(cid):
            rs = pl.ds(cid * CS, CS)
            pltpu.sync_copy(x0t.at[c, rs], b0)
            pltpu.sync_copy(xbuf.at[0, c, rs], b1)
            pltpu.sync_copy(xbuf.at[1, c, rs], b2)
            pltpu.sync_copy(acc.at[rs], b3)

            def ab(i, _):
                for h in range(0, half, 16):
                    ix = (i, pl.ds(h, 16))
                    b0[ix] = (b0[ix] + b1[ix] + b2[ix] + b3[ix]) * 0.25
                return 0
            lax.fori_loop(0, CS, ab, 0)

            @pl.when(cid < ucut)
            def _():
                pltpu.sync_copy(b0, user_o.at[c, pl.ds(cid * CS, CS)])

            @pl.when(cid >= ucut)
            def _():
                pltpu.sync_copy(b0, item_o.at[c, pl.ds(cid * CS - n_user, CS)])
        row_loop(fm)

    return k


def kernel(user_emb, item_emb, edge_weight, edge_index):
    n_user, dim = user_emb.shape
    n_item = item_emb.shape[0]
    n_nodes = n_user + n_item
    half = dim // NC
    x = jnp.concatenate([user_emb, item_emb], axis=0)
    x0f = x.reshape(n_nodes * NC, half)
    x0t = x.reshape(n_nodes, NC, half).transpose(1, 0, 2)
    zer = jnp.zeros((CS, half), jnp.float32)
    src = edge_index[1].astype(jnp.int32)
    dst = edge_index[0].astype(jnp.int32)
    w = edge_weight.astype(jnp.float32)
    n_edges = src.shape[0]
    quant = NS * ECH * 3
    n_pad = (-n_edges) % quant
    if n_pad:
        src = jnp.concatenate([src, jnp.zeros((n_pad,), jnp.int32)])
        dst = jnp.concatenate([dst, jnp.zeros((n_pad,), jnp.int32)])
        w = jnp.concatenate([w, jnp.zeros((n_pad,), jnp.float32)])
    dst2 = dst.reshape(-1, 128)
    k = _build(n_nodes, src.shape[0], half, n_user, 3)
    u, it, _ = k(x0f, x0t, zer, src, dst2, w)
    return (u.transpose(1, 0, 2).reshape(n_user, dim),
            it.transpose(1, 0, 2).reshape(n_item, dim))


# speedup vs baseline: 1.0085x; 1.0085x over previous
import functools

import jax
import jax.numpy as jnp
from jax import lax
from jax.experimental import pallas as pl
from jax.experimental.pallas import tpu as pltpu
from jax.experimental.pallas import tpu_sc as plsc

NC = 2
NS = 16
ECH = 128
CS = 40


@functools.lru_cache(maxsize=None)
def _build(n_nodes, n_edges, half, n_user, n_layers):
    edges_pt = n_edges // NS
    assert n_edges % (NS * ECH) == 0
    nfull = edges_pt // ECH
    assert n_nodes % CS == 0 and n_user % CS == 0
    nchunk = n_nodes // CS
    per = nchunk // NS
    extra = nchunk % NS
    ucut = n_user // CS
    nbuf = n_layers - 1

    mesh = plsc.VectorSubcoreMesh(
        core_axis_name="c", subcore_axis_name="s",
        num_cores=NC, num_subcores=NS)

    out_type = (
        jax.ShapeDtypeStruct((NC, n_user, half), jnp.float32),
        jax.ShapeDtypeStruct((NC, n_nodes - n_user, half), jnp.float32),
        jax.ShapeDtypeStruct((nbuf, NC, n_nodes, half), jnp.float32),
    )
    assert nfull % 3 == 0
    scratch = []
    for _ in range(3):
        scratch += [
            pltpu.VMEM((ECH,), jnp.int32),
            pltpu.VMEM((ECH // 128, 128), jnp.int32),
            pltpu.VMEM((ECH,), jnp.float32),
            pltpu.VMEM((ECH, half), jnp.float32),
            pltpu.SemaphoreType.DMA,
            pltpu.SemaphoreType.DMA,
            pltpu.SemaphoreType.DMA,
        ]
    scratch += [
        pltpu.VMEM_SHARED((n_nodes, half), jnp.float32),
        pltpu.SemaphoreType.DMA,
    ]
    scratch += [pltpu.VMEM((CS, half), jnp.float32) for _ in range(4)]

    @functools.partial(pl.kernel, out_type=out_type, mesh=mesh,
                       scratch_types=scratch,
                       compiler_params=pltpu.CompilerParams(
                           use_tc_tiling_on_sc=False))
    def k(x0f, x0t, zer, srch, dst2, wh, user_o, item_o, xbuf, *scr):
        bufs = [tuple(scr[i * 7:(i + 1) * 7]) for i in range(3)]
        (acc, sem) = scr[21:23]
        bsets = [list(scr[23:27])]
        c = lax.axis_index("c")
        s = lax.axis_index("s")
        ebase = s * edges_pt
        my_rows = per + jnp.where(s < extra, 1, 0)

        def row_loop(body):
            def rb(j, _):
                body(s + j * NS)
                return 0
            lax.fori_loop(0, my_rows, rb, 0)

        def transform_idx(sv_, size):
            def tb(i, _):
                sl = pl.ds(i * 16, 16)
                sv_[sl] = sv_[sl] * 2 + c
                return 0
            lax.fori_loop(0, size // 16, tb, 0, unroll=True)

        def scale(wv_, rv_, size):
            def sc_body(g, _):
                w16 = wv_[pl.ds(g * 16, 16)]
                for l in range(16):
                    e = g * 16 + l
                    we = w16[l]
                    for h in range(0, half, 16):
                        ix = (e, pl.ds(h, 16))
                        rv_[ix] = rv_[ix] * we
                return 0
            lax.fori_loop(0, size // 16, sc_body, 0)

        def edge_phase(layer, src_ref):
            nsub = ECH // 128

            def fire_idx(t, B):
                base = ebase + t * ECH
                pltpu.async_copy(srch.at[pl.ds(base, ECH)], B[0], B[6])
                pltpu.async_copy(dst2.at[pl.ds((ebase + t * ECH) // 128, nsub)],
                                 B[1], B[6])
                pltpu.async_copy(wh.at[pl.ds(base, ECH)], B[2], B[6])

            def wait_idx(t, B):
                base = ebase + t * ECH
                pltpu.make_async_copy(srch.at[pl.ds(base, ECH)], B[0], B[6]).wait()
                pltpu.make_async_copy(dst2.at[pl.ds((ebase + t * ECH) // 128, nsub)],
                                      B[1], B[6]).wait()
                pltpu.make_async_copy(wh.at[pl.ds(base, ECH)], B[2], B[6]).wait()

            def start_gather(B):
                if layer == 0:
                    transform_idx(B[0], ECH)
                for q in range(nsub):
                    pltpu.async_copy(src_ref.at[B[0].at[pl.ds(q * 128, 128)]],
                                     B[3].at[pl.ds(q * 128, 128)], B[4])

            def wait_gather(B):
                for q in range(nsub):
                    pltpu.make_async_copy(src_ref.at[B[0].at[pl.ds(q * 128, 128)]],
                                          B[3].at[pl.ds(q * 128, 128)], B[4]).wait()

            def fire_scatter(B):
                for q in range(nsub):
                    pltpu.async_copy(B[3].at[pl.ds(q * 128, 128)],
                                     acc.at[B[1].at[q]], B[5], add=True)

            def wait_scatter(B):
                for q in range(nsub):
                    pltpu.make_async_copy(B[3].at[pl.ds(q * 128, 128)],
                                          acc.at[B[1].at[q]], B[5]).wait()

            for t0 in range(2):
                B = bufs[t0]
                fire_idx(t0, B)
                wait_idx(t0, B)
                start_gather(B)

            def body(j, _):
                for kk in range(3):
                    B = bufs[kk]
                    B2 = bufs[(kk + 2) % 3]
                    t = 3 * j + kk
                    wait_gather(B)

                    @pl.when(t >= 1)
                    def _():
                        wait_scatter(B2)

                    @pl.when(t + 2 < nfull)
                    def _():
                        fire_idx(t + 2, B2)
                    scale(B[2], B[3], ECH)
                    fire_scatter(B)

                    @pl.when(t + 2 < nfull)
                    def _():
                        wait_idx(t + 2, B2)
                        start_gather(B2)
                return 0
            lax.fori_loop(0, nfull // 3, body, 0)
            wait_scatter(bufs[2])

        def crs(j):
            return pl.ds((s + j * NS) * CS, CS)

        row_loop(lambda cid: pltpu.sync_copy(zer, acc.at[pl.ds(cid * CS, CS)]))

        for layer in range(n_layers):
            plsc.subcore_barrier()
            edge_phase(layer, x0f if layer == 0 else xbuf.at[layer - 1, c])
            plsc.subcore_barrier()
            if layer < n_layers - 1:
                def wb(cid):
                    rs = pl.ds(cid * CS, CS)
                    pltpu.sync_copy(acc.at[rs], xbuf.at[layer, c, rs])
                    pltpu.sync_copy(zer, acc.at[rs])
                row_loop(wb)
                plsc.subcore_barrier()

        b0, b1, b2, b3 = bsets[0]

        def fm(cid):
            rs = pl.ds(cid * CS, CS)
            pltpu.sync_copy(x0t.at[c, rs], b0)
            pltpu.sync_copy(xbuf.at[0, c, rs], b1)
            pltpu.sync_copy(xbuf.at[1, c, rs], b2)
            pltpu.sync_copy(acc.at[rs], b3)

            def ab(i, _):
                for h in range(0, half, 16):
                    ix = (i, pl.ds(h, 16))
                    b0[ix] = (b0[ix] + b1[ix] + b2[ix] + b3[ix]) * 0.25
                return 0
            lax.fori_loop(0, CS, ab, 0)

            @pl.when(cid < ucut)
            def _():
                pltpu.sync_copy(b0, user_o.at[c, pl.ds(cid * CS, CS)])

            @pl.when(cid >= ucut)
            def _():
                pltpu.sync_copy(b0, item_o.at[c, pl.ds(cid * CS - n_user, CS)])
        row_loop(fm)

    return k


def kernel(user_emb, item_emb, edge_weight, edge_index):
    n_user, dim = user_emb.shape
    n_item = item_emb.shape[0]
    n_nodes = n_user + n_item
    half = dim // NC
    x = jnp.concatenate([user_emb, item_emb], axis=0)
    x0f = x.reshape(n_nodes * NC, half)
    x0t = x.reshape(n_nodes, NC, half).transpose(1, 0, 2)
    zer = jnp.zeros((CS, half), jnp.float32)
    src = edge_index[1].astype(jnp.int32)
    dst = edge_index[0].astype(jnp.int32)
    w = edge_weight.astype(jnp.float32)
    n_edges = src.shape[0]
    quant = NS * ECH * 3
    n_pad = (-n_edges) % quant
    if n_pad:
        src = jnp.concatenate([src, jnp.zeros((n_pad,), jnp.int32)])
        dst = jnp.concatenate([dst, jnp.zeros((n_pad,), jnp.int32)])
        w = jnp.concatenate([w, jnp.zeros((n_pad,), jnp.float32)])
    dst2 = dst.reshape(-1, 128)
    k = _build(n_nodes, src.shape[0], half, n_user, 3)
    u, it, _ = k(x0f, x0t, zer, src, dst2, w)
    return (u.transpose(1, 0, 2).reshape(n_user, dim),
            it.transpose(1, 0, 2).reshape(n_item, dim))


# speedup vs baseline: 1.0983x; 1.0890x over previous
import functools

import jax
import jax.numpy as jnp
from jax import lax
from jax.experimental import pallas as pl
from jax.experimental.pallas import tpu as pltpu
from jax.experimental.pallas import tpu_sc as plsc

NC = 2
NS = 16
ECH = 128
CS = 40
TAIL = 80


@functools.lru_cache(maxsize=None)
def _build(n_nodes, n_edges, half, n_user, n_layers):
    edges_pt = n_edges // NS
    assert n_edges % NS == 0
    nfull = edges_pt // ECH
    tail = edges_pt % ECH
    assert tail == TAIL and tail % 16 == 0 and tail > 0
    assert n_nodes % CS == 0 and n_user % CS == 0
    nchunk = n_nodes // CS
    per = nchunk // NS
    extra = nchunk % NS
    ucut = n_user // CS
    nbuf = n_layers - 1

    mesh = plsc.VectorSubcoreMesh(
        core_axis_name="c", subcore_axis_name="s",
        num_cores=NC, num_subcores=NS)

    out_type = (
        jax.ShapeDtypeStruct((NC, n_user, half), jnp.float32),
        jax.ShapeDtypeStruct((NC, n_nodes - n_user, half), jnp.float32),
        jax.ShapeDtypeStruct((nbuf, NC, n_nodes, half), jnp.float32),
    )
    assert nfull % 3 == 0
    scratch = []
    for _ in range(3):
        scratch += [
            pltpu.VMEM((ECH,), jnp.int32),
            pltpu.VMEM((ECH,), jnp.int32),
            pltpu.VMEM((ECH,), jnp.float32),
            pltpu.VMEM((ECH, half), jnp.float32),
            pltpu.SemaphoreType.DMA,
            pltpu.SemaphoreType.DMA,
            pltpu.SemaphoreType.DMA,
        ]
    scratch += [
        pltpu.VMEM((TAIL,), jnp.int32),
        pltpu.VMEM((TAIL,), jnp.int32),
        pltpu.VMEM((TAIL,), jnp.float32),
        pltpu.VMEM((TAIL, half), jnp.float32),
        pltpu.VMEM_SHARED((n_nodes, half), jnp.float32),
        pltpu.SemaphoreType.DMA,
    ]
    scratch += [pltpu.VMEM((CS, half), jnp.float32) for _ in range(4)]

    @functools.partial(pl.kernel, out_type=out_type, mesh=mesh,
                       scratch_types=scratch,
                       compiler_params=pltpu.CompilerParams(
                           use_tc_tiling_on_sc=False))
    def k(x0f, x0t, zer, srch, dsth, wh, user_o, item_o, xbuf, *scr):
        bufs = [tuple(scr[i * 7:(i + 1) * 7]) for i in range(3)]
        (sv2, dv2, wv2, rv2, acc, sem) = scr[21:27]
        bsets = [list(scr[27:31])]
        c = lax.axis_index("c")
        s = lax.axis_index("s")
        ebase = s * edges_pt
        my_rows = per + jnp.where(s < extra, 1, 0)

        def row_loop(body):
            def rb(j, _):
                body(s + j * NS)
                return 0
            lax.fori_loop(0, my_rows, rb, 0)

        def transform_idx(sv_, size):
            def tb(i, _):
                sl = pl.ds(i * 16, 16)
                sv_[sl] = sv_[sl] * 2 + c
                return 0
            lax.fori_loop(0, size // 16, tb, 0, unroll=True)

        def scale(wv_, rv_, size):
            def sc_body(g, _):
                w16 = wv_[pl.ds(g * 16, 16)]
                for l in range(16):
                    e = g * 16 + l
                    we = w16[l]
                    for h in range(0, half, 16):
                        ix = (e, pl.ds(h, 16))
                        rv_[ix] = rv_[ix] * we
                return 0
            lax.fori_loop(0, size // 16, sc_body, 0)

        def edge_phase(layer, src_ref):
            def fire_idx(t, B):
                base = ebase + t * ECH
                pltpu.async_copy(srch.at[pl.ds(base, ECH)], B[0], B[6])
                pltpu.async_copy(dsth.at[pl.ds(base, ECH)], B[1], B[6])
                pltpu.async_copy(wh.at[pl.ds(base, ECH)], B[2], B[6])

            def wait_idx(t, B):
                base = ebase + t * ECH
                pltpu.make_async_copy(srch.at[pl.ds(base, ECH)], B[0], B[6]).wait()
                pltpu.make_async_copy(dsth.at[pl.ds(base, ECH)], B[1], B[6]).wait()
                pltpu.make_async_copy(wh.at[pl.ds(base, ECH)], B[2], B[6]).wait()

            def start_gather(B):
                if layer == 0:
                    transform_idx(B[0], ECH)
                pltpu.async_copy(src_ref.at[B[0]], B[3], B[4])

            def wait_gather(B):
                pltpu.make_async_copy(src_ref.at[B[0]], B[3], B[4]).wait()

            def fire_scatter(B):
                pltpu.async_copy(B[3], acc.at[B[1]], B[5], add=True)

            def wait_scatter(B):
                pltpu.make_async_copy(B[3], acc.at[B[1]], B[5]).wait()

            for t0 in range(2):
                B = bufs[t0]
                fire_idx(t0, B)
                wait_idx(t0, B)
                start_gather(B)

            def body(j, _):
                for kk in range(3):
                    B = bufs[kk]
                    B2 = bufs[(kk + 2) % 3]
                    t = 3 * j + kk
                    wait_gather(B)

                    @pl.when(t >= 1)
                    def _():
                        wait_scatter(B2)

                    @pl.when(t + 2 < nfull)
                    def _():
                        fire_idx(t + 2, B2)
                    scale(B[2], B[3], ECH)
                    fire_scatter(B)

                    @pl.when(t + 2 < nfull)
                    def _():
                        wait_idx(t + 2, B2)
                        start_gather(B2)
                return 0
            lax.fori_loop(0, nfull // 3, body, 0)
            wait_scatter(bufs[2])
            base = ebase + nfull * ECH
            pltpu.sync_copy(srch.at[pl.ds(base, tail)], sv2)
            pltpu.sync_copy(dsth.at[pl.ds(base, tail)], dv2)
            pltpu.sync_copy(wh.at[pl.ds(base, tail)], wv2)
            if layer == 0:
                transform_idx(sv2, tail)
            pltpu.async_copy(src_ref.at[sv2], rv2, sem).wait()
            scale(wv2, rv2, tail)
            pltpu.sync_copy(rv2, acc.at[dv2], add=True)

        def crs(j):
            return pl.ds((s + j * NS) * CS, CS)

        row_loop(lambda cid: pltpu.sync_copy(zer, acc.at[pl.ds(cid * CS, CS)]))

        for layer in range(n_layers):
            plsc.subcore_barrier()
            edge_phase(layer, x0f if layer == 0 else xbuf.at[layer - 1, c])
            plsc.subcore_barrier()
            if layer < n_layers - 1:
                def wb(cid):
                    rs = pl.ds(cid * CS, CS)
                    pltpu.sync_copy(acc.at[rs], xbuf.at[layer, c, rs])
                    pltpu.sync_copy(zer, acc.at[rs])
                row_loop(wb)
                plsc.subcore_barrier()

        b0, b1, b2, b3 = bsets[0]

        def fm(cid):
            rs = pl.ds(cid * CS, CS)
            pltpu.sync_copy(x0t.at[c, rs], b0)
            pltpu.sync_copy(xbuf.at[0, c, rs], b1)
            pltpu.sync_copy(xbuf.at[1, c, rs], b2)
            pltpu.sync_copy(acc.at[rs], b3)

            def ab(i, _):
                for h in range(0, half, 16):
                    ix = (i, pl.ds(h, 16))
                    b0[ix] = (b0[ix] + b1[ix] + b2[ix] + b3[ix]) * 0.25
                return 0
            lax.fori_loop(0, CS, ab, 0)

            @pl.when(cid < ucut)
            def _():
                pltpu.sync_copy(b0, user_o.at[c, pl.ds(cid * CS, CS)])

            @pl.when(cid >= ucut)
            def _():
                pltpu.sync_copy(b0, item_o.at[c, pl.ds(cid * CS - n_user, CS)])
        row_loop(fm)

    return k


def kernel(user_emb, item_emb, edge_weight, edge_index):
    n_user, dim = user_emb.shape
    n_item = item_emb.shape[0]
    n_nodes = n_user + n_item
    half = dim // NC
    x = jnp.concatenate([user_emb, item_emb], axis=0)
    x0f = x.reshape(n_nodes * NC, half)
    x0t = x.reshape(n_nodes, NC, half).transpose(1, 0, 2)
    zer = jnp.zeros((CS, half), jnp.float32)
    src = edge_index[1].astype(jnp.int32)
    dst = edge_index[0].astype(jnp.int32)
    w = edge_weight.astype(jnp.float32)
    k = _build(n_nodes, src.shape[0], half, n_user, 3)
    u, it, _ = k(x0f, x0t, zer, src, dst, w)
    return (u.transpose(1, 0, 2).reshape(n_user, dim),
            it.transpose(1, 0, 2).reshape(n_item, dim))
